# baseline (speedup 1.0000x reference)
import functools

import jax
import jax.numpy as jnp
from jax import lax
from jax.experimental import pallas as pl
from jax.experimental.pallas import tpu as pltpu
from jax.experimental.pallas import tpu_sc as plsc

_N = 10000
_E = 320000
_K = 128
_NCH = _E // _K
_R = 320
_TRASH = _R
_NROW = _R + 8


def _seg(h, srcs, dsts, cinfo):
    mesh = plsc.VectorSubcoreMesh(core_axis_name="c", subcore_axis_name="s")

    @functools.partial(
        pl.kernel,
        out_type=jax.ShapeDtypeStruct((32 * _R, 128), jnp.float32),
        mesh=mesh,
        scratch_types=[
            pltpu.VMEM((16,), jnp.int32),
            pltpu.VMEM((2, _K), jnp.int32),
            pltpu.VMEM((2, _K), jnp.int32),
            pltpu.VMEM((2, _K, 128), jnp.float32),
            pltpu.VMEM((_NROW, 128), jnp.float32),
            pltpu.SemaphoreType.DMA,
        ],
        compiler_params=pltpu.CompilerParams(use_tc_tiling_on_sc=False,
                                             needs_layout_passes=False),
    )
    def seg_kernel(h_hbm, src_hbm, dst_hbm, cinfo_hbm, out_hbm,
                   cv, sidx, didx, buf, acc, gsem):
        t = lax.axis_index("c") * 16 + lax.axis_index("s")
        pltpu.sync_copy(cinfo_hbm.at[t], cv)
        cvv = cv[pl.ds(0, 16)]
        c0 = cvv[0]
        n = cvv[1]
        lo = t * _R
        zv = jnp.zeros((16,), jnp.float32)
        coli = lax.iota(jnp.int32, 16)
        cols = [coli + 16 * k for k in range(8)]

        def zero_body(r, _):
            for k in range(8):
                acc[r, pl.ds(k * 16, 16)] = zv
            return 0

        lax.fori_loop(0, _NROW, zero_body, 0)

        def stage(ch, b):
            off = pl.multiple_of((c0 + ch) * _K, _K)
            pltpu.sync_copy(src_hbm.at[pl.ds(off, _K)], sidx.at[b])
            pltpu.sync_copy(dst_hbm.at[pl.ds(off, _K)], didx.at[b])
            pltpu.async_copy(h_hbm.at[sidx.at[b]], buf.at[b], gsem)

        def chunk_body(ch, carry):
            b = ch % 2
            pltpu.make_async_copy(h_hbm.at[sidx.at[b]], buf.at[b],
                                  gsem).wait()

            @pl.when(ch + 1 < n)
            def _():
                stage(ch + 1, 1 - b)

            def group_body(g, _):
                dv = didx[b, pl.ds(g * 16, 16)]
                for j in range(16):
                    ld = dv[j] - lo
                    inb = (ld >= 0) & (ld < _R)
                    row = jnp.where(inb, ld, _TRASH)
                    rowv = jnp.full((16,), row, jnp.int32)
                    e = g * 16 + j
                    for k in range(8):
                        plsc.addupdate_scatter(
                            acc, [rowv, cols[k]],
                            buf[b, e, pl.ds(k * 16, 16)])
                return 0

            lax.fori_loop(0, _K // 16, group_body, 0)
            return carry

        @pl.when(n > 0)
        def _():
            stage(0, 0)
            lax.fori_loop(0, n, chunk_body, 0)

        pltpu.sync_copy(acc.at[pl.ds(0, _R)],
                        out_hbm.at[pl.ds(pl.multiple_of(t * _R, _R), _R)])

    return seg_kernel(h, srcs, dsts, cinfo)


def _tc_layer(agg, h, wl, wr, b, act):
    n, hin = h.shape
    hout = wl.shape[1]
    blk = 1000

    def body(a_ref, h_ref, wl_ref, wr_ref, b_ref, o_ref):
        r = (jnp.dot(a_ref[...], wl_ref[...],
                     preferred_element_type=jnp.float32)
             + jnp.dot(h_ref[...], wr_ref[...],
                       preferred_element_type=jnp.float32)
             + b_ref[...])
        o_ref[...] = jax.nn.relu(r) if act == "relu" else jnp.tanh(r)

    return pl.pallas_call(
        body,
        grid=(n // blk,),
        in_specs=[
            pl.BlockSpec((blk, hin), lambda i: (i, 0)),
            pl.BlockSpec((blk, hin), lambda i: (i, 0)),
            pl.BlockSpec((hin, hout), lambda i: (0, 0)),
            pl.BlockSpec((hin, hout), lambda i: (0, 0)),
            pl.BlockSpec((1, hout), lambda i: (0, 0)),
        ],
        out_specs=pl.BlockSpec((blk, hout), lambda i: (i, 0)),
        out_shape=jax.ShapeDtypeStruct((n, hout), jnp.float32),
    )(agg, h, wl, wr, b.reshape(1, hout))


def kernel(x, edge_index, Wl0, Wr0, b0, Wl1, Wr1, b1, Wl2, Wr2, b2,
           Wl3, Wr3, b3):
    src = edge_index[0]
    dst = edge_index[1]
    order = jnp.argsort(dst, stable=True)
    srcs = src[order]
    dsts = dst[order]
    bounds = jnp.searchsorted(dsts, jnp.arange(33, dtype=jnp.int32) * _R)
    c0 = bounds[:-1] // _K
    c1 = (bounds[1:] + _K - 1) // _K
    cinfo = jnp.zeros((32, 16), jnp.int32)
    cinfo = cinfo.at[:, 0].set(c0.astype(jnp.int32))
    cinfo = cinfo.at[:, 1].set((c1 - c0).astype(jnp.int32))

    h = x
    for wl, wr, b, act in ((Wl0, Wr0, b0, "relu"), (Wl1, Wr1, b1, "relu"),
                           (Wl2, Wr2, b2, "relu"), (Wl3, Wr3, b3, "tanh")):
        agg = _seg(h, srcs, dsts, cinfo)[:_N]
        h = _tc_layer(agg, h, wl, wr, b, act)
    return h


# speedup vs baseline: 1.0051x; 1.0051x over previous
import functools

import jax
import jax.numpy as jnp
from jax import lax
from jax.experimental import pallas as pl
from jax.experimental.pallas import tpu as pltpu
from jax.experimental.pallas import tpu_sc as plsc

_N = 10000
_E = 320000
_K = 128
_NCH = _E // _K
_R = 320
_TRASH = _R
_NROW = _R + 8


def _seg(h, srcs, dsts, cinfo):
    mesh = plsc.VectorSubcoreMesh(core_axis_name="c", subcore_axis_name="s")

    @functools.partial(
        pl.kernel,
        out_type=jax.ShapeDtypeStruct((32 * _R, 128), jnp.float32),
        mesh=mesh,
        scratch_types=[
            pltpu.VMEM((16,), jnp.int32),
            pltpu.VMEM((2, _K), jnp.int32),
            pltpu.VMEM((2, _K), jnp.int32),
            pltpu.VMEM((2, _K, 128), jnp.float32),
            pltpu.VMEM((_NROW, 128), jnp.float32),
            pltpu.SemaphoreType.DMA,
        ],
        compiler_params=pltpu.CompilerParams(use_tc_tiling_on_sc=False),
    )
    def seg_kernel(h_hbm, src_hbm, dst_hbm, cinfo_hbm, out_hbm,
                   cv, sidx, didx, buf, acc, gsem):
        t = lax.axis_index("c") * 16 + lax.axis_index("s")
        pltpu.sync_copy(cinfo_hbm.at[t], cv)
        cvv = cv[pl.ds(0, 16)]
        c0 = cvv[0]
        n = cvv[1]
        lo = t * _R
        zv = jnp.zeros((16,), jnp.float32)

        def zero_body(r, _):
            for k in range(8):
                acc[r, pl.ds(k * 16, 16)] = zv
            return 0

        lax.fori_loop(0, _NROW, zero_body, 0)

        def stage(ch, b):
            off = pl.multiple_of((c0 + ch) * _K, _K)
            pltpu.sync_copy(src_hbm.at[pl.ds(off, _K)], sidx.at[b])
            pltpu.sync_copy(dst_hbm.at[pl.ds(off, _K)], didx.at[b])
            pltpu.async_copy(h_hbm.at[sidx.at[b]], buf.at[b], gsem)

        def chunk_body(ch, carry):
            b = ch % 2
            pltpu.make_async_copy(h_hbm.at[sidx.at[b]], buf.at[b],
                                  gsem).wait()

            @pl.when(ch + 1 < n)
            def _():
                stage(ch + 1, 1 - b)

            def group_body(g, _):
                dv = didx[b, pl.ds(g * 16, 16)]
                for j in range(16):
                    ld = dv[j] - lo
                    inb = (ld >= 0) & (ld < _R)
                    row = jnp.where(inb, ld, _TRASH)
                    e = g * 16 + j
                    for k in range(8):
                        plsc.addupdate(
                            acc.at[row, pl.ds(k * 16, 16)],
                            buf[b, e, pl.ds(k * 16, 16)])
                return 0

            lax.fori_loop(0, _K // 16, group_body, 0)
            return carry

        @pl.when(n > 0)
        def _():
            stage(0, 0)
            lax.fori_loop(0, n, chunk_body, 0)

        pltpu.sync_copy(acc.at[pl.ds(0, _R)],
                        out_hbm.at[pl.ds(pl.multiple_of(t * _R, _R), _R)])

    return seg_kernel(h, srcs, dsts, cinfo)


def _tc_layer(agg, h, wl, wr, b, act):
    n, hin = h.shape
    hout = wl.shape[1]
    blk = 1000

    def body(a_ref, h_ref, wl_ref, wr_ref, b_ref, o_ref):
        r = (jnp.dot(a_ref[...], wl_ref[...],
                     preferred_element_type=jnp.float32)
             + jnp.dot(h_ref[...], wr_ref[...],
                       preferred_element_type=jnp.float32)
             + b_ref[...])
        o_ref[...] = jax.nn.relu(r) if act == "relu" else jnp.tanh(r)

    return pl.pallas_call(
        body,
        grid=(n // blk,),
        in_specs=[
            pl.BlockSpec((blk, hin), lambda i: (i, 0)),
            pl.BlockSpec((blk, hin), lambda i: (i, 0)),
            pl.BlockSpec((hin, hout), lambda i: (0, 0)),
            pl.BlockSpec((hin, hout), lambda i: (0, 0)),
            pl.BlockSpec((1, hout), lambda i: (0, 0)),
        ],
        out_specs=pl.BlockSpec((blk, hout), lambda i: (i, 0)),
        out_shape=jax.ShapeDtypeStruct((n, hout), jnp.float32),
    )(agg, h, wl, wr, b.reshape(1, hout))


def kernel(x, edge_index, Wl0, Wr0, b0, Wl1, Wr1, b1, Wl2, Wr2, b2,
           Wl3, Wr3, b3):
    src = edge_index[0]
    dst = edge_index[1]
    order = jnp.argsort(dst, stable=True)
    srcs = src[order]
    dsts = dst[order]
    bounds = jnp.searchsorted(dsts, jnp.arange(33, dtype=jnp.int32) * _R)
    c0 = bounds[:-1] // _K
    c1 = (bounds[1:] + _K - 1) // _K
    cinfo = jnp.zeros((32, 16), jnp.int32)
    cinfo = cinfo.at[:, 0].set(c0.astype(jnp.int32))
    cinfo = cinfo.at[:, 1].set((c1 - c0).astype(jnp.int32))

    h = x
    for wl, wr, b, act in ((Wl0, Wr0, b0, "relu"), (Wl1, Wr1, b1, "relu"),
                           (Wl2, Wr2, b2, "relu"), (Wl3, Wr3, b3, "tanh")):
        agg = _seg(h, srcs, dsts, cinfo)[:_N]
        h = _tc_layer(agg, h, wl, wr, b, act)
    return h
